# baseline (speedup 1.0000x reference)
import functools
import jax
import jax.numpy as jnp
from jax import lax
from jax.experimental import pallas as pl
from jax.experimental.pallas import tpu as pltpu
from jax.experimental.pallas import tpu_sc as plsc

DIM = 64
VOCAB = 1000000
B = 4096
S = 200
TOTAL = B * S
NC, NS = 2, 16
NW = NC * NS
LANES = 16

COLS_FULL = VOCAB // 128
TAIL = VOCAB - COLS_FULL * 128


@functools.partial(
    pl.kernel,
    out_type=jax.ShapeDtypeStruct((VOCAB * DIM,), jnp.float32),
    mesh=plsc.VectorSubcoreMesh(core_axis_name="c", subcore_axis_name="s"),
    scratch_types=[
        pltpu.VMEM((DIM, 128), jnp.float32),
        pltpu.VMEM((DIM, 128), jnp.float32),
        pltpu.VMEM((DIM * 128,), jnp.float32),
        pltpu.VMEM((DIM * 128,), jnp.float32),
        pltpu.VMEM((DIM, TAIL), jnp.float32),
        pltpu.VMEM((DIM * TAIL,), jnp.float32),
        pltpu.SemaphoreType.DMA,
        pltpu.SemaphoreType.DMA,
    ],
    compiler_params=pltpu.CompilerParams(needs_layout_passes=False),
)
def _table_transpose(tt_hbm, tlin_hbm, blk_v0, blk_v1, tout_v0, tout_v1,
                     blk_t, tout_t, in_sem, out_sem):
    wid = lax.axis_index("s") * NC + lax.axis_index("c")
    iota64 = lax.broadcasted_iota(jnp.int32, (LANES,), 0) * DIM
    blk = (blk_v0, blk_v1)
    tout = (tout_v0, tout_v1)

    def in_copy(c, bf):
        return pltpu.make_async_copy(
            tt_hbm.at[:, pl.ds(c * 128, 128)], blk[bf], in_sem)

    def out_copy(c, bf):
        return pltpu.make_async_copy(
            tout[bf], tlin_hbm.at[pl.ds(c * DIM * 128, DIM * 128)], out_sem)

    def transpose_blk(src, dst, width):
        @pl.loop(0, DIM, step=4)
        def _d(d0):
            for dd in range(4):
                d = d0 + dd
                for q in range(width // LANES):
                    v = src[d, pl.ds(q * LANES, LANES)]
                    idx = iota64 + (q * LANES * DIM + d)
                    plsc.store_scatter(dst, [idx], v)

    in_copy(wid, 0).start()

    @pl.loop(0, 246, step=2)
    def _g(g):
        for u in range(2):
            i = g + u
            c = wid + i * NW

            @pl.when(c < COLS_FULL)
            def _():
                in_copy(c, u).wait()

                @pl.when(c + NW < COLS_FULL)
                def _():
                    in_copy(c + NW, 1 - u).start()

                @pl.when(i >= 2)
                def _():
                    out_copy(c - 2 * NW, u).wait()

                transpose_blk(blk[u], tout[u], 128)
                out_copy(c, u).start()

    n_cols = jnp.where(wid < COLS_FULL - (COLS_FULL // NW) * NW,
                       COLS_FULL // NW + 1, COLS_FULL // NW)
    out_copy(wid + (n_cols - 1) * NW, 0).wait()
    out_copy(wid + (n_cols - 2) * NW, 0).wait()

    @pl.when(wid == NW - 1)
    def _():
        pltpu.sync_copy(tt_hbm.at[:, pl.ds(COLS_FULL * 128, TAIL)], blk_t)
        transpose_blk(blk_t, tout_t, TAIL)
        pltpu.sync_copy(
            tout_t, tlin_hbm.at[pl.ds(COLS_FULL * 128 * DIM, TAIL * DIM)])


N_UNIT = S * (B // 128)
UNITS_PER_W = N_UNIT // NW


@functools.partial(
    pl.kernel,
    out_type=jax.ShapeDtypeStruct((TOTAL * DIM,), jnp.float32),
    mesh=plsc.VectorSubcoreMesh(core_axis_name="c", subcore_axis_name="s"),
    scratch_types=[
        pltpu.VMEM((128,), jnp.int32),
        pltpu.VMEM((128,), jnp.int32),
        pltpu.VMEM((128, DIM), jnp.float32),
        pltpu.VMEM((128, DIM), jnp.float32),
        pltpu.VMEM((DIM * 128,), jnp.float32),
        pltpu.VMEM((DIM * 128,), jnp.float32),
        pltpu.SemaphoreType.DMA,
        pltpu.SemaphoreType.DMA,
        pltpu.SemaphoreType.DMA,
    ],
    compiler_params=pltpu.CompilerParams(
        use_tc_tiling_on_sc=False, needs_layout_passes=False),
)
def _gather_tiled(idx_hbm, table_hbm, out_hbm, idx_v0, idx_v1,
                  rows_v0, rows_v1, t_v0, t_v1, idx_sem, gat_sem, out_sem):
    wid = lax.axis_index("s") * NC + lax.axis_index("c")
    iota128 = lax.broadcasted_iota(jnp.int32, (LANES,), 0) * 128
    idxb = (idx_v0, idx_v1)
    rows = (rows_v0, rows_v1)
    tv = (t_v0, t_v1)

    def idx_copy(s, bf):
        return pltpu.make_async_copy(
            idx_hbm.at[pl.ds(s * B + wid * 128, 128)], idxb[bf], idx_sem)

    def gather(bf):
        return pltpu.make_async_copy(
            table_hbm.at[idxb[bf]], rows[bf], gat_sem)

    def out_copies(s, bf):
        base = s * (DIM * B) + wid * 1024
        return [
            pltpu.make_async_copy(
                tv[bf].at[pl.ds(r * 1024, 1024)],
                out_hbm.at[pl.ds(base + r * (B // 128) * 1024, 1024)],
                out_sem)
            for r in range(DIM // 8)
        ]

    def transpose_rows(bf):
        @pl.loop(0, 128, step=4)
        def _b(b0):
            for bb in range(4):
                b = b0 + bb
                for q in range(DIM // LANES):
                    v = rows[bf][b, pl.ds(q * LANES, LANES)]
                    idx = iota128 + (q * LANES * 128 + b)
                    plsc.store_scatter(tv[bf], [idx], v)

    idx_copy(0, 0).start()
    idx_copy(0, 0).wait()
    gather(0).start()
    idx_copy(1, 1).start()

    @pl.loop(0, UNITS_PER_W, step=2)
    def _g(g):
        for u in range(2):
            i = g + u
            gather(u).wait()

            @pl.when(i + 1 < UNITS_PER_W)
            def _():
                idx_copy(i + 1, 1 - u).wait()
                gather(1 - u).start()

            @pl.when(i + 2 < UNITS_PER_W)
            def _():
                idx_copy(i + 2, u).start()

            @pl.when(i >= 2)
            def _():
                for d in out_copies(i - 2, u):
                    d.wait()

            transpose_rows(u)
            for d in out_copies(i, u):
                d.start()

    for i in (UNITS_PER_W - 2, UNITS_PER_W - 1):
        for d in out_copies(i, i % 2):
            d.wait()


def kernel(x, embedding):
    tlin = _table_transpose(embedding.T)
    table = tlin.reshape(VOCAB, DIM)
    idx = x.T.reshape(TOTAL)
    out1d = _gather_tiled(idx, table)
    a = out1d.reshape(S, DIM // 8, B // 128, 8, 128)
    b = a.transpose(2, 4, 0, 1, 3)
    return b.reshape(B, S, DIM)


# speedup vs baseline: 1.4056x; 1.4056x over previous
import functools
import jax
import jax.numpy as jnp
from jax import lax
from jax.experimental import pallas as pl
from jax.experimental.pallas import tpu as pltpu
from jax.experimental.pallas import tpu_sc as plsc

DIM = 64
VOCAB = 1000000
B = 4096
S = 200
TOTAL = B * S
NC, NS = 2, 16
NW = NC * NS
LANES = 16

COLS_FULL = VOCAB // 128
TAIL = VOCAB - COLS_FULL * 128


@functools.partial(
    pl.kernel,
    out_type=jax.ShapeDtypeStruct((VOCAB * DIM,), jnp.float32),
    mesh=plsc.VectorSubcoreMesh(core_axis_name="c", subcore_axis_name="s"),
    scratch_types=[
        pltpu.VMEM((DIM, 128), jnp.float32),
        pltpu.VMEM((DIM, 128), jnp.float32),
        pltpu.VMEM((DIM * 128,), jnp.float32),
        pltpu.VMEM((DIM * 128,), jnp.float32),
        pltpu.VMEM((DIM, TAIL), jnp.float32),
        pltpu.VMEM((DIM * TAIL,), jnp.float32),
        pltpu.SemaphoreType.DMA,
        pltpu.SemaphoreType.DMA,
    ],
    compiler_params=pltpu.CompilerParams(needs_layout_passes=False),
)
def _table_transpose(tt_hbm, tlin_hbm, blk_v0, blk_v1, tout_v0, tout_v1,
                     blk_t, tout_t, in_sem, out_sem):
    wid = lax.axis_index("s") * NC + lax.axis_index("c")
    iota64 = lax.broadcasted_iota(jnp.int32, (LANES,), 0) * DIM
    blk = (blk_v0, blk_v1)
    tout = (tout_v0, tout_v1)

    def in_copy(c, bf):
        return pltpu.make_async_copy(
            tt_hbm.at[:, pl.ds(c * 128, 128)], blk[bf], in_sem)

    def out_copy(c, bf):
        return pltpu.make_async_copy(
            tout[bf], tlin_hbm.at[pl.ds(c * DIM * 128, DIM * 128)], out_sem)

    def transpose_blk(src, dst, width):
        @plsc.parallel_loop(0, DIM, unroll=8)
        def _d(d):
            for q in range(width // LANES):
                v = src[d, pl.ds(q * LANES, LANES)]
                idx = iota64 + (q * LANES * DIM + d)
                plsc.store_scatter(dst, [idx], v)

    in_copy(wid, 0).start()

    @pl.loop(0, 246, step=2)
    def _g(g):
        for u in range(2):
            i = g + u
            c = wid + i * NW

            @pl.when(c < COLS_FULL)
            def _():
                in_copy(c, u).wait()

                @pl.when(c + NW < COLS_FULL)
                def _():
                    in_copy(c + NW, 1 - u).start()

                @pl.when(i >= 2)
                def _():
                    out_copy(c - 2 * NW, u).wait()

                transpose_blk(blk[u], tout[u], 128)
                out_copy(c, u).start()

    n_cols = jnp.where(wid < COLS_FULL - (COLS_FULL // NW) * NW,
                       COLS_FULL // NW + 1, COLS_FULL // NW)
    out_copy(wid + (n_cols - 1) * NW, 0).wait()
    out_copy(wid + (n_cols - 2) * NW, 0).wait()

    @pl.when(wid == NW - 1)
    def _():
        pltpu.sync_copy(tt_hbm.at[:, pl.ds(COLS_FULL * 128, TAIL)], blk_t)
        transpose_blk(blk_t, tout_t, TAIL)
        pltpu.sync_copy(
            tout_t, tlin_hbm.at[pl.ds(COLS_FULL * 128 * DIM, TAIL * DIM)])


N_UNIT = S * (B // 128)
UNITS_PER_W = N_UNIT // NW


@functools.partial(
    pl.kernel,
    out_type=jax.ShapeDtypeStruct((TOTAL * DIM,), jnp.float32),
    mesh=plsc.VectorSubcoreMesh(core_axis_name="c", subcore_axis_name="s"),
    scratch_types=[
        pltpu.VMEM((128,), jnp.int32),
        pltpu.VMEM((128,), jnp.int32),
        pltpu.VMEM((128, DIM), jnp.float32),
        pltpu.VMEM((128, DIM), jnp.float32),
        pltpu.VMEM((DIM * 128,), jnp.float32),
        pltpu.VMEM((DIM * 128,), jnp.float32),
        pltpu.SemaphoreType.DMA,
        pltpu.SemaphoreType.DMA,
        pltpu.SemaphoreType.DMA,
    ],
    compiler_params=pltpu.CompilerParams(
        use_tc_tiling_on_sc=False, needs_layout_passes=False),
)
def _gather_tiled(idx_hbm, table_hbm, out_hbm, idx_v0, idx_v1,
                  rows_v0, rows_v1, t_v0, t_v1, idx_sem, gat_sem, out_sem):
    wid = lax.axis_index("s") * NC + lax.axis_index("c")
    iota128 = lax.broadcasted_iota(jnp.int32, (LANES,), 0) * 128
    idxb = (idx_v0, idx_v1)
    rows = (rows_v0, rows_v1)
    tv = (t_v0, t_v1)

    def idx_copy(s, bf):
        return pltpu.make_async_copy(
            idx_hbm.at[pl.ds(s * B + wid * 128, 128)], idxb[bf], idx_sem)

    def gather(bf):
        return pltpu.make_async_copy(
            table_hbm.at[idxb[bf]], rows[bf], gat_sem)

    def out_copies(s, bf):
        base = s * (DIM * B) + wid * 1024
        return [
            pltpu.make_async_copy(
                tv[bf].at[pl.ds(r * 1024, 1024)],
                out_hbm.at[pl.ds(base + r * (B // 128) * 1024, 1024)],
                out_sem)
            for r in range(DIM // 8)
        ]

    def transpose_rows(bf):
        @plsc.parallel_loop(0, 128, unroll=8)
        def _b(b):
            for q in range(DIM // LANES):
                v = rows[bf][b, pl.ds(q * LANES, LANES)]
                idx = iota128 + (q * LANES * 128 + b)
                plsc.store_scatter(tv[bf], [idx], v)

    idx_copy(0, 0).start()
    idx_copy(0, 0).wait()
    gather(0).start()
    idx_copy(1, 1).start()

    @pl.loop(0, UNITS_PER_W, step=2)
    def _g(g):
        for u in range(2):
            i = g + u
            gather(u).wait()

            @pl.when(i + 1 < UNITS_PER_W)
            def _():
                idx_copy(i + 1, 1 - u).wait()
                gather(1 - u).start()

            @pl.when(i + 2 < UNITS_PER_W)
            def _():
                idx_copy(i + 2, u).start()

            @pl.when(i >= 2)
            def _():
                for d in out_copies(i - 2, u):
                    d.wait()

            transpose_rows(u)
            for d in out_copies(i, u):
                d.start()

    for i in (UNITS_PER_W - 2, UNITS_PER_W - 1):
        for d in out_copies(i, i % 2):
            d.wait()


def kernel(x, embedding):
    tlin = _table_transpose(embedding.T)
    table = tlin.reshape(VOCAB, DIM)
    idx = x.T.reshape(TOTAL)
    out1d = _gather_tiled(idx, table)
    a = out1d.reshape(S, DIM // 8, B // 128, 8, 128)
    b = a.transpose(2, 4, 0, 1, 3)
    return b.reshape(B, S, DIM)


# speedup vs baseline: 2.2560x; 1.6050x over previous
import functools
import jax
import jax.numpy as jnp
from jax import lax
from jax.experimental import pallas as pl
from jax.experimental.pallas import tpu as pltpu
from jax.experimental.pallas import tpu_sc as plsc

DIM = 64
VOCAB = 1000000
B = 4096
S = 200
TOTAL = B * S
NC, NS = 2, 16
NW = NC * NS
LANES = 16


def _diag_transpose(src, dst, A, width):
    iota = lax.broadcasted_iota(jnp.int32, (LANES,), 0)
    rot = [(iota + k) & (LANES - 1) for k in range(LANES)]
    rot_a = [r * A + iota for r in rot]

    @plsc.parallel_loop(0, A // LANES)
    def _ai(ai):
        base_row = iota + ai * LANES
        for bj in range(width // LANES):
            for k in range(LANES):
                v = plsc.load_gather(src, [base_row, rot[k] + bj * LANES])
                plsc.store_scatter(
                    dst, [rot_a[k] + (bj * LANES * A + ai * LANES)], v)

COLS_FULL = VOCAB // 128
TAIL = VOCAB - COLS_FULL * 128


@functools.partial(
    pl.kernel,
    out_type=jax.ShapeDtypeStruct((VOCAB * DIM,), jnp.float32),
    mesh=plsc.VectorSubcoreMesh(core_axis_name="c", subcore_axis_name="s"),
    scratch_types=[
        pltpu.VMEM((DIM, 128), jnp.float32),
        pltpu.VMEM((DIM, 128), jnp.float32),
        pltpu.VMEM((DIM * 128,), jnp.float32),
        pltpu.VMEM((DIM * 128,), jnp.float32),
        pltpu.VMEM((DIM, TAIL), jnp.float32),
        pltpu.VMEM((DIM * TAIL,), jnp.float32),
        pltpu.SemaphoreType.DMA,
        pltpu.SemaphoreType.DMA,
    ],
    compiler_params=pltpu.CompilerParams(needs_layout_passes=False),
)
def _table_transpose(tt_hbm, tlin_hbm, blk_v0, blk_v1, tout_v0, tout_v1,
                     blk_t, tout_t, in_sem, out_sem):
    wid = lax.axis_index("s") * NC + lax.axis_index("c")
    blk = (blk_v0, blk_v1)
    tout = (tout_v0, tout_v1)

    def in_copy(c, bf):
        return pltpu.make_async_copy(
            tt_hbm.at[:, pl.ds(c * 128, 128)], blk[bf], in_sem)

    def out_copy(c, bf):
        return pltpu.make_async_copy(
            tout[bf], tlin_hbm.at[pl.ds(c * DIM * 128, DIM * 128)], out_sem)

    def transpose_blk(src, dst, width):
        _diag_transpose(src, dst, DIM, width)

    in_copy(wid, 0).start()

    @pl.loop(0, 246, step=2)
    def _g(g):
        for u in range(2):
            i = g + u
            c = wid + i * NW

            @pl.when(c < COLS_FULL)
            def _():
                in_copy(c, u).wait()

                @pl.when(c + NW < COLS_FULL)
                def _():
                    in_copy(c + NW, 1 - u).start()

                @pl.when(i >= 2)
                def _():
                    out_copy(c - 2 * NW, u).wait()

                transpose_blk(blk[u], tout[u], 128)
                out_copy(c, u).start()

    n_cols = jnp.where(wid < COLS_FULL - (COLS_FULL // NW) * NW,
                       COLS_FULL // NW + 1, COLS_FULL // NW)
    out_copy(wid + (n_cols - 1) * NW, 0).wait()
    out_copy(wid + (n_cols - 2) * NW, 0).wait()

    @pl.when(wid == NW - 1)
    def _():
        pltpu.sync_copy(tt_hbm.at[:, pl.ds(COLS_FULL * 128, TAIL)], blk_t)
        transpose_blk(blk_t, tout_t, TAIL)
        pltpu.sync_copy(
            tout_t, tlin_hbm.at[pl.ds(COLS_FULL * 128 * DIM, TAIL * DIM)])


N_UNIT = S * (B // 128)
UNITS_PER_W = N_UNIT // NW


@functools.partial(
    pl.kernel,
    out_type=jax.ShapeDtypeStruct((TOTAL * DIM,), jnp.float32),
    mesh=plsc.VectorSubcoreMesh(core_axis_name="c", subcore_axis_name="s"),
    scratch_types=[
        pltpu.VMEM((128,), jnp.int32),
        pltpu.VMEM((128,), jnp.int32),
        pltpu.VMEM((128, DIM), jnp.float32),
        pltpu.VMEM((128, DIM), jnp.float32),
        pltpu.VMEM((DIM * 128,), jnp.float32),
        pltpu.VMEM((DIM * 128,), jnp.float32),
        pltpu.SemaphoreType.DMA,
        pltpu.SemaphoreType.DMA,
        pltpu.SemaphoreType.DMA,
    ],
    compiler_params=pltpu.CompilerParams(
        use_tc_tiling_on_sc=False, needs_layout_passes=False),
)
def _gather_tiled(idx_hbm, table_hbm, out_hbm, idx_v0, idx_v1,
                  rows_v0, rows_v1, t_v0, t_v1, idx_sem, gat_sem, out_sem):
    wid = lax.axis_index("s") * NC + lax.axis_index("c")
    idxb = (idx_v0, idx_v1)
    rows = (rows_v0, rows_v1)
    tv = (t_v0, t_v1)

    def idx_copy(s, bf):
        return pltpu.make_async_copy(
            idx_hbm.at[pl.ds(s * B + wid * 128, 128)], idxb[bf], idx_sem)

    def gather(bf):
        return pltpu.make_async_copy(
            table_hbm.at[idxb[bf]], rows[bf], gat_sem)

    def out_copies(s, bf):
        base = s * (DIM * B) + wid * 1024
        return [
            pltpu.make_async_copy(
                tv[bf].at[pl.ds(r * 1024, 1024)],
                out_hbm.at[pl.ds(base + r * (B // 128) * 1024, 1024)],
                out_sem)
            for r in range(DIM // 8)
        ]

    def transpose_rows(bf):
        _diag_transpose(rows[bf], tv[bf], 128, DIM)

    idx_copy(0, 0).start()
    idx_copy(0, 0).wait()
    gather(0).start()
    idx_copy(1, 1).start()

    @pl.loop(0, UNITS_PER_W, step=2)
    def _g(g):
        for u in range(2):
            i = g + u
            gather(u).wait()

            @pl.when(i + 1 < UNITS_PER_W)
            def _():
                idx_copy(i + 1, 1 - u).wait()
                gather(1 - u).start()

            @pl.when(i + 2 < UNITS_PER_W)
            def _():
                idx_copy(i + 2, u).start()

            @pl.when(i >= 2)
            def _():
                for d in out_copies(i - 2, u):
                    d.wait()

            transpose_rows(u)
            for d in out_copies(i, u):
                d.start()

    for i in (UNITS_PER_W - 2, UNITS_PER_W - 1):
        for d in out_copies(i, i % 2):
            d.wait()


def kernel(x, embedding):
    tlin = _table_transpose(embedding.T)
    table = tlin.reshape(VOCAB, DIM)
    idx = x.T.reshape(TOTAL)
    out1d = _gather_tiled(idx, table)
    a = out1d.reshape(S, DIM // 8, B // 128, 8, 128)
    b = a.transpose(2, 4, 0, 1, 3)
    return b.reshape(B, S, DIM)


# speedup vs baseline: 2.5813x; 1.1442x over previous
import functools
import jax
import jax.numpy as jnp
from jax import lax
from jax.experimental import pallas as pl
from jax.experimental.pallas import tpu as pltpu
from jax.experimental.pallas import tpu_sc as plsc

DIM = 64
VOCAB = 1000000
B = 4096
S = 200
TOTAL = B * S
NC, NS = 2, 16
NW = NC * NS
LANES = 16


def _diag_transpose(src, dst, A, width, dyn_bj=False):
    iota = lax.broadcasted_iota(jnp.int32, (LANES,), 0)
    rot = [(iota + k) & (LANES - 1) for k in range(LANES)]
    rot_a = [r * A + iota for r in rot]

    def block(base_row, ai, bj):
        for k in range(LANES):
            v = plsc.load_gather(src, [base_row, rot[k] + bj * LANES])
            plsc.store_scatter(
                dst, [rot_a[k] + (bj * (LANES * A) + ai * LANES)], v)

    @plsc.parallel_loop(0, A // LANES)
    def _ai(ai):
        base_row = iota + ai * LANES
        if dyn_bj:
            @pl.loop(0, width // LANES)
            def _bj(bj):
                block(base_row, ai, bj)
        else:
            for bj in range(width // LANES):
                block(base_row, ai, bj)

CW = 256
COLS_FULL = VOCAB // CW
TAIL = VOCAB - COLS_FULL * CW
MAX_I = -(-COLS_FULL // NW)


@functools.partial(
    pl.kernel,
    out_type=jax.ShapeDtypeStruct((VOCAB * DIM,), jnp.float32),
    mesh=plsc.VectorSubcoreMesh(core_axis_name="c", subcore_axis_name="s"),
    scratch_types=[
        pltpu.VMEM((DIM, CW), jnp.float32),
        pltpu.VMEM((DIM, CW), jnp.float32),
        pltpu.VMEM((DIM * CW,), jnp.float32),
        pltpu.VMEM((DIM * CW,), jnp.float32),
        pltpu.VMEM((DIM, TAIL), jnp.float32),
        pltpu.VMEM((DIM * TAIL,), jnp.float32),
        pltpu.SemaphoreType.DMA,
        pltpu.SemaphoreType.DMA,
    ],
    compiler_params=pltpu.CompilerParams(needs_layout_passes=False),
)
def _table_transpose(tt_hbm, tlin_hbm, blk_v0, blk_v1, tout_v0, tout_v1,
                     blk_t, tout_t, in_sem, out_sem):
    wid = lax.axis_index("s") * NC + lax.axis_index("c")
    blk = (blk_v0, blk_v1)
    tout = (tout_v0, tout_v1)

    def in_copy(c, bf):
        return pltpu.make_async_copy(
            tt_hbm.at[:, pl.ds(c * CW, CW)], blk[bf], in_sem)

    def out_copy(c, bf):
        return pltpu.make_async_copy(
            tout[bf], tlin_hbm.at[pl.ds(c * DIM * CW, DIM * CW)], out_sem)

    in_copy(wid, 0).start()

    @pl.loop(0, MAX_I + 1, step=2)
    def _g(g):
        for u in range(2):
            i = g + u
            c = wid + i * NW

            @pl.when(c < COLS_FULL)
            def _():
                in_copy(c, u).wait()

                @pl.when(c + NW < COLS_FULL)
                def _():
                    in_copy(c + NW, 1 - u).start()

                @pl.when(i >= 2)
                def _():
                    out_copy(c - 2 * NW, u).wait()

                _diag_transpose(blk[u], tout[u], DIM, CW, dyn_bj=True)
                out_copy(c, u).start()

    n_cols = jnp.where(wid < COLS_FULL - (COLS_FULL // NW) * NW,
                       COLS_FULL // NW + 1, COLS_FULL // NW)
    out_copy(wid + (n_cols - 1) * NW, 0).wait()
    out_copy(wid + (n_cols - 2) * NW, 0).wait()

    @pl.when(wid == NW - 1)
    def _():
        pltpu.sync_copy(tt_hbm.at[:, pl.ds(COLS_FULL * CW, TAIL)], blk_t)
        _diag_transpose(blk_t, tout_t, DIM, TAIL)
        pltpu.sync_copy(
            tout_t, tlin_hbm.at[pl.ds(COLS_FULL * CW * DIM, TAIL * DIM)])


N_UNIT = S * (B // 128)
UNITS_PER_W = N_UNIT // NW


@functools.partial(
    pl.kernel,
    out_type=jax.ShapeDtypeStruct((TOTAL * DIM,), jnp.float32),
    mesh=plsc.VectorSubcoreMesh(core_axis_name="c", subcore_axis_name="s"),
    scratch_types=[
        pltpu.VMEM((S, 128), jnp.int32),
        pltpu.VMEM((128, DIM), jnp.float32),
        pltpu.VMEM((128, DIM), jnp.float32),
        pltpu.VMEM((DIM * 128,), jnp.float32),
        pltpu.VMEM((DIM * 128,), jnp.float32),
        pltpu.SemaphoreType.DMA,
        pltpu.SemaphoreType.DMA,
    ],
    compiler_params=pltpu.CompilerParams(
        use_tc_tiling_on_sc=False, needs_layout_passes=False),
)
def _gather_tiled(idx_hbm, table_hbm, out_hbm, idx_all,
                  rows_v0, rows_v1, t_v0, t_v1, gat_sem, out_sem):
    wid = lax.axis_index("s") * NC + lax.axis_index("c")
    rows = (rows_v0, rows_v1)
    tv = (t_v0, t_v1)

    def gather(s, bf):
        return pltpu.make_async_copy(
            table_hbm.at[idx_all.at[s]], rows[bf], gat_sem)

    def out_copies(s, bf):
        base = s * (DIM * B) + wid * 1024
        return [
            pltpu.make_async_copy(
                tv[bf].at[pl.ds(r * 1024, 1024)],
                out_hbm.at[pl.ds(base + r * (B // 128) * 1024, 1024)],
                out_sem)
            for r in range(DIM // 8)
        ]

    def transpose_rows(bf):
        _diag_transpose(rows[bf], tv[bf], 128, DIM)

    pltpu.sync_copy(idx_hbm.at[:, pl.ds(wid * 128, 128)], idx_all)
    gather(0, 0).start()

    @pl.loop(0, UNITS_PER_W, step=2)
    def _g(g):
        for u in range(2):
            i = g + u
            gather(i, u).wait()

            @pl.when(i + 1 < UNITS_PER_W)
            def _():
                gather(i + 1, 1 - u).start()

            @pl.when(i >= 2)
            def _():
                for d in out_copies(i - 2, u):
                    d.wait()

            transpose_rows(u)
            for d in out_copies(i, u):
                d.start()

    for i in (UNITS_PER_W - 2, UNITS_PER_W - 1):
        for d in out_copies(i, i % 2):
            d.wait()


def kernel(x, embedding):
    tlin = _table_transpose(embedding.T)
    table = tlin.reshape(VOCAB, DIM)
    out1d = _gather_tiled(x.T, table)
    a = out1d.reshape(S, DIM // 8, B // 128, 8, 128)
    b = a.transpose(2, 4, 0, 1, 3)
    return b.reshape(B, S, DIM)


# speedup vs baseline: 2.6840x; 1.0398x over previous
import functools
import jax
import jax.numpy as jnp
from jax import lax
from jax.experimental import pallas as pl
from jax.experimental.pallas import tpu as pltpu
from jax.experimental.pallas import tpu_sc as plsc

DIM = 64
VOCAB = 1000000
B = 4096
S = 200
TOTAL = B * S
NC, NS = 2, 16
NW = NC * NS
LANES = 16


def _diag_transpose(src, dst, A, width, dyn_bj=False):
    iota = lax.broadcasted_iota(jnp.int32, (LANES,), 0)
    rot = [(iota + k) & (LANES - 1) for k in range(LANES)]
    rot_a = [r * A + iota for r in rot]

    def block(base_row, ai, bj):
        for k in range(LANES):
            v = plsc.load_gather(src, [base_row, rot[k] + bj * LANES])
            plsc.store_scatter(
                dst, [rot_a[k] + (bj * (LANES * A) + ai * LANES)], v)

    @plsc.parallel_loop(0, A // LANES)
    def _ai(ai):
        base_row = iota + ai * LANES
        if dyn_bj:
            @pl.loop(0, width // LANES, step=4)
            def _bj(bj0):
                for j in range(4):
                    block(base_row, ai, bj0 + j)
        else:
            for bj in range(width // LANES):
                block(base_row, ai, bj)

CW = 256
COLS_FULL = VOCAB // CW
TAIL = VOCAB - COLS_FULL * CW
MAX_I = -(-COLS_FULL // NW)


@functools.partial(
    pl.kernel,
    out_type=jax.ShapeDtypeStruct((VOCAB * DIM,), jnp.float32),
    mesh=plsc.VectorSubcoreMesh(core_axis_name="c", subcore_axis_name="s"),
    scratch_types=[
        pltpu.VMEM((DIM, CW), jnp.float32),
        pltpu.VMEM((DIM, CW), jnp.float32),
        pltpu.VMEM((DIM * CW,), jnp.float32),
        pltpu.VMEM((DIM * CW,), jnp.float32),
        pltpu.VMEM((DIM, TAIL), jnp.float32),
        pltpu.VMEM((DIM * TAIL,), jnp.float32),
        pltpu.SemaphoreType.DMA,
        pltpu.SemaphoreType.DMA,
    ],
    compiler_params=pltpu.CompilerParams(needs_layout_passes=False),
)
def _table_transpose(tt_hbm, tlin_hbm, blk_v0, blk_v1, tout_v0, tout_v1,
                     blk_t, tout_t, in_sem, out_sem):
    wid = lax.axis_index("s") * NC + lax.axis_index("c")
    blk = (blk_v0, blk_v1)
    tout = (tout_v0, tout_v1)

    def in_copy(c, bf):
        return pltpu.make_async_copy(
            tt_hbm.at[:, pl.ds(c * CW, CW)], blk[bf], in_sem)

    def out_copy(c, bf):
        return pltpu.make_async_copy(
            tout[bf], tlin_hbm.at[pl.ds(c * DIM * CW, DIM * CW)], out_sem)

    in_copy(wid, 0).start()

    @pl.loop(0, MAX_I + 1, step=2)
    def _g(g):
        for u in range(2):
            i = g + u
            c = wid + i * NW

            @pl.when(c < COLS_FULL)
            def _():
                in_copy(c, u).wait()

                @pl.when(c + NW < COLS_FULL)
                def _():
                    in_copy(c + NW, 1 - u).start()

                @pl.when(i >= 2)
                def _():
                    out_copy(c - 2 * NW, u).wait()

                _diag_transpose(blk[u], tout[u], DIM, CW, dyn_bj=True)
                out_copy(c, u).start()

    n_cols = jnp.where(wid < COLS_FULL - (COLS_FULL // NW) * NW,
                       COLS_FULL // NW + 1, COLS_FULL // NW)
    out_copy(wid + (n_cols - 1) * NW, 0).wait()
    out_copy(wid + (n_cols - 2) * NW, 0).wait()

    @pl.when(wid == NW - 1)
    def _():
        pltpu.sync_copy(tt_hbm.at[:, pl.ds(COLS_FULL * CW, TAIL)], blk_t)
        _diag_transpose(blk_t, tout_t, DIM, TAIL)
        pltpu.sync_copy(
            tout_t, tlin_hbm.at[pl.ds(COLS_FULL * CW * DIM, TAIL * DIM)])


N_UNIT = S * (B // 128)
UNITS_PER_W = N_UNIT // NW


@functools.partial(
    pl.kernel,
    out_type=jax.ShapeDtypeStruct((TOTAL * DIM,), jnp.float32),
    mesh=plsc.VectorSubcoreMesh(core_axis_name="c", subcore_axis_name="s"),
    scratch_types=[
        pltpu.VMEM((S, 128), jnp.int32),
        pltpu.VMEM((128, DIM), jnp.float32),
        pltpu.VMEM((128, DIM), jnp.float32),
        pltpu.VMEM((DIM * 128,), jnp.float32),
        pltpu.VMEM((DIM * 128,), jnp.float32),
        pltpu.SemaphoreType.DMA,
        pltpu.SemaphoreType.DMA,
    ],
    compiler_params=pltpu.CompilerParams(
        use_tc_tiling_on_sc=False, needs_layout_passes=False),
)
def _gather_tiled(idx_hbm, table_hbm, out_hbm, idx_all,
                  rows_v0, rows_v1, t_v0, t_v1, gat_sem, out_sem):
    wid = lax.axis_index("s") * NC + lax.axis_index("c")
    rows = (rows_v0, rows_v1)
    tv = (t_v0, t_v1)

    def gather(s, bf):
        return pltpu.make_async_copy(
            table_hbm.at[idx_all.at[s]], rows[bf], gat_sem)

    def out_copies(s, bf):
        base = s * (DIM * B) + wid * 1024
        return [
            pltpu.make_async_copy(
                tv[bf].at[pl.ds(r * 1024, 1024)],
                out_hbm.at[pl.ds(base + r * (B // 128) * 1024, 1024)],
                out_sem)
            for r in range(DIM // 8)
        ]

    def transpose_rows(bf):
        _diag_transpose(rows[bf], tv[bf], 128, DIM)

    pltpu.sync_copy(idx_hbm.at[:, pl.ds(wid * 128, 128)], idx_all)
    gather(0, 0).start()

    @pl.loop(0, UNITS_PER_W, step=2)
    def _g(g):
        for u in range(2):
            i = g + u
            gather(i, u).wait()

            @pl.when(i + 1 < UNITS_PER_W)
            def _():
                gather(i + 1, 1 - u).start()

            @pl.when(i >= 2)
            def _():
                for d in out_copies(i - 2, u):
                    d.wait()

            transpose_rows(u)
            for d in out_copies(i, u):
                d.start()

    for i in (UNITS_PER_W - 2, UNITS_PER_W - 1):
        for d in out_copies(i, i % 2):
            d.wait()


def kernel(x, embedding):
    tlin = _table_transpose(embedding.T)
    table = tlin.reshape(VOCAB, DIM)
    out1d = _gather_tiled(x.T, table)
    a = out1d.reshape(S, DIM // 8, B // 128, 8, 128)
    b = a.transpose(2, 4, 0, 1, 3)
    return b.reshape(B, S, DIM)
